# baseline (speedup 1.0000x reference)
import functools

import jax
import jax.numpy as jnp
from jax import lax
from jax.experimental import pallas as pl
from jax.experimental.pallas import tpu as pltpu
from jax.experimental.pallas import tpu_sc as plsc

_NC = 2
_NS = 16
_NW = _NC * _NS
_L = 16


def _sc_neighbor_ids(inputs, nb_table):
    B = inputs.shape[0]
    NB = nb_table.shape[1]
    IPW = B // _NW

    mesh = plsc.VectorSubcoreMesh(core_axis_name="c", subcore_axis_name="s")

    @functools.partial(
        pl.kernel,
        out_type=jax.ShapeDtypeStruct((B, NB), jnp.int32),
        mesh=mesh,
        compiler_params=pltpu.CompilerParams(use_tc_tiling_on_sc=False),
        scratch_types=[
            pltpu.VMEM((IPW,), jnp.int32),
            pltpu.VMEM((IPW, NB), jnp.int32),
            pltpu.SemaphoreType.DMA,
        ],
    )
    def kern(ids_hbm, nb_hbm, out_hbm, idx_v, nbr_v, sem):
        wid = lax.axis_index("s") * _NC + lax.axis_index("c")
        base = wid * IPW
        pltpu.sync_copy(ids_hbm.at[pl.ds(base, IPW)], idx_v)
        pltpu.async_copy(nb_hbm.at[idx_v], nbr_v, sem).wait()
        pltpu.sync_copy(nbr_v, out_hbm.at[pl.ds(base, IPW)])

    return kern(inputs, nb_table)


def _sc_gather_top2(inputs, nbrs_flat, feat_table, half, nhalf):
    B = inputs.shape[0]
    D = feat_table.shape[1]
    NB = 128 * nbrs_flat.shape[0] // B
    BH = B // nhalf
    G = D // _L
    IPW = BH // _NW
    P = 128 // NB
    NGRP = IPW // P
    CH = 64
    NCHUNK = IPW // CH
    GPC = CH // P
    NBUF = 4

    mesh = plsc.VectorSubcoreMesh(core_axis_name="c", subcore_axis_name="s")

    @functools.partial(
        pl.kernel,
        out_type=(
            jax.ShapeDtypeStruct((BH, D), jnp.float32),
            jax.ShapeDtypeStruct((BH, D), jnp.float32),
            jax.ShapeDtypeStruct((BH, D), jnp.float32),
        ),
        mesh=mesh,
        scratch_types=[
            pltpu.VMEM((IPW,), jnp.int32),
            pltpu.VMEM((NGRP, P * NB), jnp.int32),
            pltpu.VMEM((CH, D), jnp.float32),
            pltpu.VMEM((CH, D), jnp.float32),
            pltpu.VMEM((CH, D), jnp.float32),
            [pltpu.VMEM((P * NB, D), jnp.float32)] * NBUF,
            pltpu.SemaphoreType.DMA,
            [pltpu.SemaphoreType.DMA] * NBUF,
        ],
    )
    def kern(ids_hbm, nbf_hbm, feat_hbm, nf_hbm, t1_hbm, t2_hbm,
             idx_v, nbr_v, nf_v, t1_v, t2_v, rows, sem_nf, sem_r):
        wid = lax.axis_index("s") * _NC + lax.axis_index("c")
        base = wid * IPW
        pltpu.sync_copy(ids_hbm.at[pl.ds(half * BH + base, IPW)], idx_v)
        pltpu.sync_copy(
            nbf_hbm.at[pl.ds((half * BH // P) + wid * NGRP, NGRP)], nbr_v)

        def start_rows(grp, buf, sem):
            pltpu.async_copy(feat_hbm.at[nbr_v.at[grp]], buf, sem)

        def wait_rows(buf, sem):
            pltpu.make_async_copy(feat_hbm.at[nbr_v.at[0]], buf, sem).wait()

        def reduce_group(buf, tbase):
            def item_body(p, carry):
                for g in range(G):
                    sl = pl.ds(g * _L, _L)
                    r0 = buf[p * NB + 0, sl]
                    r1 = buf[p * NB + 1, sl]
                    m1 = jnp.maximum(r0, r1)
                    m2 = jnp.minimum(r0, r1)
                    for j in range(2, NB):
                        v = buf[p * NB + j, sl]
                        m2 = jnp.maximum(m2, jnp.minimum(m1, v))
                        m1 = jnp.maximum(m1, v)
                    t1_v[tbase + p, sl] = m1
                    t2_v[tbase + p, sl] = m2
                return carry

            lax.fori_loop(0, P, item_body, 0)

        for b in range(NBUF):
            start_rows(b, rows[b], sem_r[b])

        def chunk_body(c, carry):
            cb = c * CH
            nf_cp = pltpu.async_copy(
                feat_hbm.at[idx_v.at[pl.ds(cb, CH)]], nf_v, sem_nf)

            def ring_body(k, carry2):
                g0 = c * GPC + NBUF * k
                for b in range(NBUF):
                    g = g0 + b
                    wait_rows(rows[b], sem_r[b])
                    reduce_group(rows[b], (NBUF * k + b) * P)
                    start_rows(jnp.minimum(g + NBUF, NGRP - 1),
                               rows[b], sem_r[b])
                return carry2

            lax.fori_loop(0, GPC // NBUF, ring_body, 0)
            nf_cp.wait()
            pltpu.sync_copy(nf_v, nf_hbm.at[pl.ds(base + cb, CH)])
            pltpu.sync_copy(t1_v, t1_hbm.at[pl.ds(base + cb, CH)])
            pltpu.sync_copy(t2_v, t2_hbm.at[pl.ds(base + cb, CH)])
            return carry

        lax.fori_loop(0, NCHUNK, chunk_body, 0)
        for b in range(NBUF):
            wait_rows(rows[b], sem_r[b])

    return kern(inputs, nbrs_flat, feat_table)


def _tc_matmuls(nf, t1, t2, W1, b1, W2, b2):
    B, D = nf.shape
    H = W1.shape[2]
    OUT = W2.shape[2]
    BLK = 1024

    def body(nf_ref, t1_ref, t2_ref, w1_ref, b1_ref, w2_ref, b2_ref, o_ref):
        x0 = nf_ref[...]
        x1 = t1_ref[...]
        x2 = t2_ref[...]
        w10 = w1_ref[0]
        w11 = w1_ref[1]
        dot = functools.partial(jnp.dot, preferred_element_type=jnp.float32)
        h0 = dot(x0, w10) + dot(x1, w11) + b1_ref[...]
        h1 = dot(x1, w10) + dot(x2, w11) + b1_ref[...]
        o_ref[...] = dot(h0, w2_ref[0]) + dot(h1, w2_ref[1]) + b2_ref[...]

    return pl.pallas_call(
        body,
        grid=(B // BLK,),
        in_specs=[
            pl.BlockSpec((BLK, D), lambda i: (i, 0)),
            pl.BlockSpec((BLK, D), lambda i: (i, 0)),
            pl.BlockSpec((BLK, D), lambda i: (i, 0)),
            pl.BlockSpec((2, D, H), lambda i: (0, 0, 0)),
            pl.BlockSpec((1, H), lambda i: (0, 0)),
            pl.BlockSpec((2, H, OUT), lambda i: (0, 0, 0)),
            pl.BlockSpec((1, OUT), lambda i: (0, 0)),
        ],
        out_specs=pl.BlockSpec((BLK, OUT), lambda i: (i, 0)),
        out_shape=jax.ShapeDtypeStruct((B, OUT), jnp.float32),
    )(nf, t1, t2, W1, b1.reshape(1, H), W2, b2.reshape(1, OUT))


def kernel(inputs, nb_table, feat_table, W1, b1, W2, b2):
    B = inputs.shape[0]
    NB = nb_table.shape[1]
    N = nb_table.shape[0]
    nbrs_flat = (jax.lax.broadcasted_iota(jnp.int32, (B * NB // 128, 128), 0)
                 * 131 % N)
    nf, t1, t2 = _sc_gather_top2(inputs, nbrs_flat, feat_table, 0, 1)
    return _tc_matmuls(nf, t1, t2, W1, b1, W2, b2)


# speedup vs baseline: 2.7709x; 2.7709x over previous
import functools

import jax
import jax.numpy as jnp
from jax import lax
from jax.experimental import pallas as pl
from jax.experimental.pallas import tpu as pltpu
from jax.experimental.pallas import tpu_sc as plsc

_NC = 2
_NS = 16
_NW = _NC * _NS
_L = 16


def _sc_neighbor_ids(inputs, nb_table):
    B = inputs.shape[0]
    NB = nb_table.shape[1]
    IPW = B // _NW

    mesh = plsc.VectorSubcoreMesh(core_axis_name="c", subcore_axis_name="s")

    @functools.partial(
        pl.kernel,
        out_type=jax.ShapeDtypeStruct((B, NB), jnp.int32),
        mesh=mesh,
        compiler_params=pltpu.CompilerParams(use_tc_tiling_on_sc=False),
        scratch_types=[
            pltpu.VMEM((IPW,), jnp.int32),
            pltpu.VMEM((IPW, NB), jnp.int32),
            pltpu.SemaphoreType.DMA,
        ],
    )
    def kern(ids_hbm, nb_hbm, out_hbm, idx_v, nbr_v, sem):
        wid = lax.axis_index("s") * _NC + lax.axis_index("c")
        base = wid * IPW
        pltpu.sync_copy(ids_hbm.at[pl.ds(base, IPW)], idx_v)
        pltpu.async_copy(nb_hbm.at[idx_v], nbr_v, sem).wait()
        pltpu.sync_copy(nbr_v, out_hbm.at[pl.ds(base, IPW)])

    return kern(inputs, nb_table)


def _sc_gather_top2(inputs, nbrs_flat, feat_table):
    B = inputs.shape[0]
    D = feat_table.shape[1]
    NB = 128 * nbrs_flat.shape[0] // B
    G = D // _L
    IPW = B // _NW
    P = 128 // NB
    NGRP = IPW // P
    CH = 64
    NCHUNK = IPW // CH
    GPC = CH // P
    NBUF = 4

    mesh = plsc.VectorSubcoreMesh(core_axis_name="c", subcore_axis_name="s")

    @functools.partial(
        pl.kernel,
        out_type=(
            jax.ShapeDtypeStruct((B, D), jnp.float32),
            jax.ShapeDtypeStruct((B, D), jnp.float32),
            jax.ShapeDtypeStruct((B, D), jnp.float32),
        ),
        mesh=mesh,
        scratch_types=[
            pltpu.VMEM((IPW,), jnp.int32),
            pltpu.VMEM((NGRP, P * NB), jnp.int32),
            pltpu.VMEM((CH, D), jnp.float32),
            pltpu.VMEM((CH, D), jnp.float32),
            pltpu.VMEM((CH, D), jnp.float32),
            [pltpu.VMEM((P * NB, D), jnp.float32)] * NBUF,
            pltpu.SemaphoreType.DMA,
            [pltpu.SemaphoreType.DMA] * NBUF,
        ],
    )
    def kern(ids_hbm, nbf_hbm, feat_hbm, nf_hbm, t1_hbm, t2_hbm,
             idx_v, nbr_v, nf_v, t1_v, t2_v, rows, sem_nf, sem_r):
        wid = lax.axis_index("s") * _NC + lax.axis_index("c")
        base = wid * IPW
        pltpu.sync_copy(ids_hbm.at[pl.ds(base, IPW)], idx_v)
        pltpu.sync_copy(nbf_hbm.at[pl.ds(wid * NGRP, NGRP)], nbr_v)

        def start_rows(grp, buf, sem):
            pltpu.async_copy(feat_hbm.at[nbr_v.at[grp]], buf, sem)

        def wait_rows(buf, sem):
            pltpu.make_async_copy(feat_hbm.at[nbr_v.at[0]], buf, sem).wait()

        def reduce_group(buf, tbase):
            def item_body(p, carry):
                for g in range(G):
                    sl = pl.ds(g * _L, _L)
                    r0 = buf[p * NB + 0, sl]
                    r1 = buf[p * NB + 1, sl]
                    m1 = jnp.maximum(r0, r1)
                    m2 = jnp.minimum(r0, r1)
                    for j in range(2, NB):
                        v = buf[p * NB + j, sl]
                        m2 = jnp.maximum(m2, jnp.minimum(m1, v))
                        m1 = jnp.maximum(m1, v)
                    t1_v[tbase + p, sl] = m1
                    t2_v[tbase + p, sl] = m2
                return carry

            lax.fori_loop(0, P, item_body, 0)

        for b in range(NBUF):
            start_rows(b, rows[b], sem_r[b])

        def chunk_body(c, carry):
            cb = c * CH
            nf_cp = pltpu.async_copy(
                feat_hbm.at[idx_v.at[pl.ds(cb, CH)]], nf_v, sem_nf)

            def ring_body(k, carry2):
                g0 = c * GPC + NBUF * k
                for b in range(NBUF):
                    g = g0 + b
                    wait_rows(rows[b], sem_r[b])
                    reduce_group(rows[b], (NBUF * k + b) * P)
                    start_rows(jnp.minimum(g + NBUF, NGRP - 1),
                               rows[b], sem_r[b])
                return carry2

            lax.fori_loop(0, GPC // NBUF, ring_body, 0)
            nf_cp.wait()
            pltpu.sync_copy(nf_v, nf_hbm.at[pl.ds(base + cb, CH)])
            pltpu.sync_copy(t1_v, t1_hbm.at[pl.ds(base + cb, CH)])
            pltpu.sync_copy(t2_v, t2_hbm.at[pl.ds(base + cb, CH)])
            return carry

        lax.fori_loop(0, NCHUNK, chunk_body, 0)
        for b in range(NBUF):
            wait_rows(rows[b], sem_r[b])

    return kern(inputs, nbrs_flat, feat_table)


def _tc_matmuls(nf, t1, t2, W1, b1, W2, b2):
    B, D = nf.shape
    H = W1.shape[2]
    OUT = W2.shape[2]
    BLK = 1024

    def body(nf_ref, t1_ref, t2_ref, w1_ref, b1_ref, w2_ref, b2_ref, o_ref):
        x0 = nf_ref[...]
        x1 = t1_ref[...]
        x2 = t2_ref[...]
        w10 = w1_ref[0]
        w11 = w1_ref[1]
        dot = functools.partial(jnp.dot, preferred_element_type=jnp.float32)
        h0 = dot(x0, w10) + dot(x1, w11) + b1_ref[...]
        h1 = dot(x1, w10) + dot(x2, w11) + b1_ref[...]
        o_ref[...] = dot(h0, w2_ref[0]) + dot(h1, w2_ref[1]) + b2_ref[...]

    return pl.pallas_call(
        body,
        grid=(B // BLK,),
        in_specs=[
            pl.BlockSpec((BLK, D), lambda i: (i, 0)),
            pl.BlockSpec((BLK, D), lambda i: (i, 0)),
            pl.BlockSpec((BLK, D), lambda i: (i, 0)),
            pl.BlockSpec((2, D, H), lambda i: (0, 0, 0)),
            pl.BlockSpec((1, H), lambda i: (0, 0)),
            pl.BlockSpec((2, H, OUT), lambda i: (0, 0, 0)),
            pl.BlockSpec((1, OUT), lambda i: (0, 0)),
        ],
        out_specs=pl.BlockSpec((BLK, OUT), lambda i: (i, 0)),
        out_shape=jax.ShapeDtypeStruct((B, OUT), jnp.float32),
    )(nf, t1, t2, W1, b1.reshape(1, H), W2, b2.reshape(1, OUT))


def kernel(inputs, nb_table, feat_table, W1, b1, W2, b2):
    B = inputs.shape[0]
    NB = nb_table.shape[1]
    nbrs = _sc_neighbor_ids(inputs, nb_table)
    nbrs_flat = nbrs.reshape(B * NB // 128, 128)
    nf, t1, t2 = _sc_gather_top2(inputs, nbrs_flat, feat_table)
    return _tc_matmuls(nf, t1, t2, W1, b1, W2, b2)


# speedup vs baseline: 2.8031x; 1.0116x over previous
import functools

import jax
import jax.numpy as jnp
from jax import lax
from jax.experimental import pallas as pl
from jax.experimental.pallas import tpu as pltpu
from jax.experimental.pallas import tpu_sc as plsc

_NC = 2
_NS = 16
_NW = _NC * _NS
_L = 16


def _sc_neighbor_ids(inputs, nb_table):
    B = inputs.shape[0]
    NB = nb_table.shape[1]
    IPW = B // _NW

    mesh = plsc.VectorSubcoreMesh(core_axis_name="c", subcore_axis_name="s")

    @functools.partial(
        pl.kernel,
        out_type=jax.ShapeDtypeStruct((B, NB), jnp.int32),
        mesh=mesh,
        compiler_params=pltpu.CompilerParams(use_tc_tiling_on_sc=False),
        scratch_types=[
            pltpu.VMEM((IPW,), jnp.int32),
            pltpu.VMEM((IPW, NB), jnp.int32),
            pltpu.SemaphoreType.DMA,
        ],
    )
    def kern(ids_hbm, nb_hbm, out_hbm, idx_v, nbr_v, sem):
        wid = lax.axis_index("s") * _NC + lax.axis_index("c")
        base = wid * IPW
        pltpu.sync_copy(ids_hbm.at[pl.ds(base, IPW)], idx_v)
        pltpu.async_copy(nb_hbm.at[idx_v], nbr_v, sem).wait()
        pltpu.sync_copy(nbr_v, out_hbm.at[pl.ds(base, IPW)])

    return kern(inputs, nb_table)


def _sc_gather_top2(inputs, nbrs_flat, feat_table):
    B = inputs.shape[0]
    D = feat_table.shape[1]
    NB = 128 * nbrs_flat.shape[0] // B
    G = D // _L
    IPW = B // _NW
    P = 128 // NB
    NGRP = IPW // P
    CH = 64
    NCHUNK = IPW // CH
    GPC = CH // P
    NBUF = 4

    mesh = plsc.VectorSubcoreMesh(core_axis_name="c", subcore_axis_name="s")

    @functools.partial(
        pl.kernel,
        out_type=(
            jax.ShapeDtypeStruct((B, D), jnp.float32),
            jax.ShapeDtypeStruct((B, D), jnp.float32),
            jax.ShapeDtypeStruct((B, D), jnp.float32),
        ),
        mesh=mesh,
        scratch_types=[
            pltpu.VMEM((IPW,), jnp.int32),
            pltpu.VMEM((NGRP, P * NB), jnp.int32),
            pltpu.VMEM((CH, D), jnp.float32),
            pltpu.VMEM((CH, D), jnp.float32),
            pltpu.VMEM((CH, D), jnp.float32),
            [pltpu.VMEM((P * NB, D), jnp.float32)] * NBUF,
            pltpu.SemaphoreType.DMA,
            [pltpu.SemaphoreType.DMA] * NBUF,
        ],
    )
    def kern(ids_hbm, nbf_hbm, feat_hbm, nf_hbm, t1_hbm, t2_hbm,
             idx_v, nbr_v, nf_v, t1_v, t2_v, rows, sem_nf, sem_r):
        wid = lax.axis_index("s") * _NC + lax.axis_index("c")
        base = wid * IPW
        pltpu.sync_copy(ids_hbm.at[pl.ds(base, IPW)], idx_v)
        pltpu.sync_copy(nbf_hbm.at[pl.ds(wid * NGRP, NGRP)], nbr_v)

        def start_rows(grp, buf, sem):
            pltpu.async_copy(feat_hbm.at[nbr_v.at[grp]], buf, sem)

        def wait_rows(buf, sem):
            pltpu.make_async_copy(feat_hbm.at[nbr_v.at[0]], buf, sem).wait()

        def reduce_group(buf, tbase):
            def item_body(p, carry):
                for g in range(G):
                    sl = pl.ds(g * _L, _L)
                    r0 = buf[p * NB + 0, sl]
                    r1 = buf[p * NB + 1, sl]
                    m1 = jnp.maximum(r0, r1)
                    m2 = jnp.minimum(r0, r1)
                    for j in range(2, NB):
                        v = buf[p * NB + j, sl]
                        m2 = jnp.maximum(m2, jnp.minimum(m1, v))
                        m1 = jnp.maximum(m1, v)
                    t1_v[tbase + p, sl] = m1
                    t2_v[tbase + p, sl] = m2
                return carry

            lax.fori_loop(0, P, item_body, 0)

        for b in range(NBUF):
            start_rows(b, rows[b], sem_r[b])

        def chunk_body(c, carry):
            cb = c * CH
            nf_cp = pltpu.async_copy(
                feat_hbm.at[idx_v.at[pl.ds(cb, CH)]], nf_v, sem_nf)

            def ring_body(k, carry2):
                g0 = c * GPC + NBUF * k
                for b in range(NBUF):
                    g = g0 + b
                    wait_rows(rows[b], sem_r[b])
                    reduce_group(rows[b], (NBUF * k + b) * P)
                    start_rows(jnp.minimum(g + NBUF, NGRP - 1),
                               rows[b], sem_r[b])
                return carry2

            lax.fori_loop(0, GPC // NBUF, ring_body, 0)
            nf_cp.wait()
            pltpu.sync_copy(nf_v, nf_hbm.at[pl.ds(base + cb, CH)])
            pltpu.sync_copy(t1_v, t1_hbm.at[pl.ds(base + cb, CH)])
            pltpu.sync_copy(t2_v, t2_hbm.at[pl.ds(base + cb, CH)])
            return carry

        lax.fori_loop(0, NCHUNK, chunk_body, 0)
        for b in range(NBUF):
            wait_rows(rows[b], sem_r[b])

    return kern(inputs, nbrs_flat, feat_table)


def _tc_matmuls(nf, t1, t2, W1, b1, W2, b2):
    B, D = nf.shape
    H = W1.shape[2]
    OUT = W2.shape[2]
    BLK = 1024

    def body(nf_ref, t1_ref, t2_ref, w1_ref, b1_ref, w2_ref, b2_ref, o_ref):
        dot = functools.partial(jnp.dot, preferred_element_type=jnp.float32)
        w10 = w1_ref[0]
        w11 = w1_ref[1]
        w20 = w2_ref[0]
        w21 = w2_ref[1]
        wa = dot(w10, w20)
        wb = dot(w11, w20) + dot(w10, w21)
        wc = dot(w11, w21)
        bias = dot(b1_ref[...], w20) + dot(b1_ref[...], w21) + b2_ref[...]
        o_ref[...] = (dot(nf_ref[...], wa) + dot(t1_ref[...], wb)
                      + dot(t2_ref[...], wc) + bias)

    return pl.pallas_call(
        body,
        grid=(B // BLK,),
        in_specs=[
            pl.BlockSpec((BLK, D), lambda i: (i, 0)),
            pl.BlockSpec((BLK, D), lambda i: (i, 0)),
            pl.BlockSpec((BLK, D), lambda i: (i, 0)),
            pl.BlockSpec((2, D, H), lambda i: (0, 0, 0)),
            pl.BlockSpec((1, H), lambda i: (0, 0)),
            pl.BlockSpec((2, H, OUT), lambda i: (0, 0, 0)),
            pl.BlockSpec((1, OUT), lambda i: (0, 0)),
        ],
        out_specs=pl.BlockSpec((BLK, OUT), lambda i: (i, 0)),
        out_shape=jax.ShapeDtypeStruct((B, OUT), jnp.float32),
    )(nf, t1, t2, W1, b1.reshape(1, H), W2, b2.reshape(1, OUT))


def kernel(inputs, nb_table, feat_table, W1, b1, W2, b2):
    B = inputs.shape[0]
    NB = nb_table.shape[1]
    nbrs = _sc_neighbor_ids(inputs, nb_table)
    nbrs_flat = nbrs.reshape(B * NB // 128, 128)
    nf, t1, t2 = _sc_gather_top2(inputs, nbrs_flat, feat_table)
    return _tc_matmuls(nf, t1, t2, W1, b1, W2, b2)
